# Initial kernel scaffold; baseline (speedup 1.0000x reference)
import jax
import jax.numpy as jnp
from jax.experimental import pallas as pl


def kernel(x, edge_index, W1, b1, Wm, bm, W2, b2, Wg, att_src, att_dst, bg, Wout, bout):
    raise NotImplementedError("write your pallas kernel here")


# speedup vs baseline: 1.3422x; 1.3422x over previous
import functools
import jax
import jax.numpy as jnp
from jax.experimental import pallas as pl
from jax.experimental.pallas import tpu as pltpu

_N = 10000
_E = 160000
_HEADS = 8
_FH = 64


def _mm_kernel(x_ref, w_ref, b_ref, o_ref, *, act):
    acc = jnp.dot(x_ref[...], w_ref[...], preferred_element_type=jnp.float32)
    acc = acc + b_ref[...]
    if act == "relu":
        acc = jnp.maximum(acc, 0.0)
    o_ref[...] = acc


def _matmul(x, w, b, act="none", block_m=400):
    m, k = x.shape
    k2, n = w.shape
    grid = (m // block_m,)
    return pl.pallas_call(
        functools.partial(_mm_kernel, act=act),
        grid=grid,
        in_specs=[
            pl.BlockSpec((block_m, k), lambda i: (i, 0)),
            pl.BlockSpec((k, n), lambda i: (0, 0)),
            pl.BlockSpec((1, n), lambda i: (0, 0)),
        ],
        out_specs=pl.BlockSpec((block_m, n), lambda i: (i, 0)),
        out_shape=jax.ShapeDtypeStruct((m, n), jnp.float32),
    )(x, w, b.reshape(1, n))


def _gcn_prop(y, src, dst, dis):
    acc = jax.ops.segment_sum(y[src], dst, num_segments=_N)
    return dis[:, None] * (acc + y)


def kernel(x, edge_index, W1, b1, Wm, bm, W2, b2, Wg, att_src, att_dst, bg, Wout, bout):
    src = edge_index[0]
    dst = edge_index[1]
    ones = jnp.ones((_E,), dtype=jnp.float32)
    deg = jax.ops.segment_sum(ones, dst, num_segments=_N) + 1.0
    dis = jax.lax.rsqrt(deg)

    h = _matmul(x, W1, jnp.zeros_like(b1))
    h = _gcn_prop(dis[:, None] * h, src, dst, dis)
    h = jnp.maximum(h + b1, 0.0)
    h2 = _matmul(h, Wm, jnp.zeros_like(bm))
    h2 = _gcn_prop(dis[:, None] * h2, src, dst, dis)
    h2 = jnp.maximum(h2 + bm, 0.0)
    h3 = _matmul(h2, W2, jnp.zeros_like(b2))
    h3 = _gcn_prop(dis[:, None] * h3, src, dst, dis)
    h3 = jnp.maximum(h3 + b2, 0.0)

    g = _matmul(h3, Wg, jnp.zeros_like(bg)).reshape(_N, _HEADS, _FH)
    a_src = (g * att_src[None]).sum(-1)
    a_dst = (g * att_dst[None]).sum(-1)
    gmax = a_src.max(axis=0)
    c = a_dst + gmax[None]
    c = jnp.where(c > 0, c, 0.2 * c)

    alpha = a_src[src] + a_dst[dst]
    alpha = jnp.where(alpha > 0, alpha, 0.2 * alpha)
    ex = jnp.exp(alpha - c[dst])
    denom = jax.ops.segment_sum(ex, dst, num_segments=_N)
    aself = a_src + a_dst
    aself = jnp.where(aself > 0, aself, 0.2 * aself)
    ex_self = jnp.exp(aself - c)
    denom = denom + ex_self + 1e-16

    msg = jax.ops.segment_sum(g[src] * ex[:, :, None], dst, num_segments=_N)
    gat = (msg + g * ex_self[:, :, None]) / denom[:, :, None]
    gat = gat.reshape(_N, _HEADS * _FH) + bg

    h4 = h3 + jnp.where(gat > 0, gat, jnp.expm1(gat))

    logits = _matmul(h4, Wout, bout)
    return jax.nn.log_softmax(logits, axis=1)


# speedup vs baseline: 4.1697x; 3.1067x over previous
import functools
import jax
import jax.numpy as jnp
from jax import lax
from jax.experimental import pallas as pl
from jax.experimental.pallas import tpu as pltpu
from jax.experimental.pallas import tpu_sc as plsc

_N = 10000
_NPAD = 10112
_E = 160000
_EPAD = 163840
_HEADS = 8
_FH = 64
_H = 512
_NCHUNK = 4
_DH = 5056
_DACC = 5120
_DRPT = _DACC // 16
_BM = 400


def _mm_scale_body(x_ref, w_ref, s_ref, o_ref):
    acc = jnp.dot(x_ref[...], w_ref[...], preferred_element_type=jnp.float32)
    o_ref[...] = acc * s_ref[...]


def _tc_scale_mm(x, w, scale):
    m, k = x.shape
    n = w.shape[1]
    return pl.pallas_call(
        _mm_scale_body,
        grid=(m // _BM,),
        in_specs=[
            pl.BlockSpec((_BM, k), lambda i: (i, 0)),
            pl.BlockSpec((k, n), lambda i: (0, 0)),
            pl.BlockSpec((_BM, 1), lambda i: (i, 0)),
        ],
        out_specs=pl.BlockSpec((_BM, n), lambda i: (i, 0)),
        out_shape=jax.ShapeDtypeStruct((m, n), jnp.float32),
    )(x, w, scale)


def _chain_body(acc_ref, y_ref, s_ref, b_ref, w_ref, o_ref, h_ref):
    h = jnp.maximum(s_ref[...] * (acc_ref[...] + y_ref[...]) + b_ref[...], 0.0)
    h_ref[...] = h
    o_ref[...] = s_ref[...] * jnp.dot(h, w_ref[...],
                                      preferred_element_type=jnp.float32)


def _tc_chain(acc, y, scale, b, w):
    m, k = y.shape
    n = w.shape[1]
    return pl.pallas_call(
        _chain_body,
        grid=(m // _BM,),
        in_specs=[
            pl.BlockSpec((_BM, k), lambda i: (i, 0)),
            pl.BlockSpec((_BM, k), lambda i: (i, 0)),
            pl.BlockSpec((_BM, 1), lambda i: (i, 0)),
            pl.BlockSpec((1, k), lambda i: (0, 0)),
            pl.BlockSpec((k, n), lambda i: (0, 0)),
        ],
        out_specs=[
            pl.BlockSpec((_BM, n), lambda i: (i, 0)),
            pl.BlockSpec((_BM, k), lambda i: (i, 0)),
        ],
        out_shape=[
            jax.ShapeDtypeStruct((m, n), jnp.float32),
            jax.ShapeDtypeStruct((m, k), jnp.float32),
        ],
    )(acc, y, scale, b.reshape(1, k), w)


def _gatprep_body(acc_ref, y_ref, s_ref, b_ref, wg_ref, asf_ref, adf_ref,
                  sel_ref, h_ref, g_ref, as_ref, ad_ref):
    h = jnp.maximum(s_ref[...] * (acc_ref[...] + y_ref[...]) + b_ref[...], 0.0)
    h_ref[...] = h
    g = jnp.dot(h, wg_ref[...], preferred_element_type=jnp.float32)
    g_ref[...] = g
    as_ref[...] = jnp.dot(g * asf_ref[...], sel_ref[...],
                          preferred_element_type=jnp.float32)
    ad_ref[...] = jnp.dot(g * adf_ref[...], sel_ref[...],
                          preferred_element_type=jnp.float32)


def _tc_gat_prep(acc, y, scale, b, wg, att_src_flat, att_dst_flat, sel):
    m, k = y.shape
    return pl.pallas_call(
        _gatprep_body,
        grid=(m // _BM,),
        in_specs=[
            pl.BlockSpec((_BM, k), lambda i: (i, 0)),
            pl.BlockSpec((_BM, k), lambda i: (i, 0)),
            pl.BlockSpec((_BM, 1), lambda i: (i, 0)),
            pl.BlockSpec((1, k), lambda i: (0, 0)),
            pl.BlockSpec((k, _H), lambda i: (0, 0)),
            pl.BlockSpec((1, _H), lambda i: (0, 0)),
            pl.BlockSpec((1, _H), lambda i: (0, 0)),
            pl.BlockSpec((_H, _HEADS), lambda i: (0, 0)),
        ],
        out_specs=[
            pl.BlockSpec((_BM, k), lambda i: (i, 0)),
            pl.BlockSpec((_BM, _H), lambda i: (i, 0)),
            pl.BlockSpec((_BM, _HEADS), lambda i: (i, 0)),
            pl.BlockSpec((_BM, _HEADS), lambda i: (i, 0)),
        ],
        out_shape=[
            jax.ShapeDtypeStruct((m, k), jnp.float32),
            jax.ShapeDtypeStruct((m, _H), jnp.float32),
            jax.ShapeDtypeStruct((m, _HEADS), jnp.float32),
            jax.ShapeDtypeStruct((m, _HEADS), jnp.float32),
        ],
    )(acc, y, scale, b.reshape(1, k), wg, att_src_flat, att_dst_flat, sel)


def _final_body(msg_ref, g_ref, h3_ref, exs_ref, den_ref, selt_ref, bg_ref,
                wo_ref, bo_ref, o_ref):
    exf = jnp.dot(exs_ref[...], selt_ref[...],
                  preferred_element_type=jnp.float32)
    denf = jnp.dot(den_ref[...], selt_ref[...],
                   preferred_element_type=jnp.float32)
    gat = (msg_ref[...] + g_ref[...] * exf) / denf + bg_ref[...]
    elu = jnp.where(gat > 0, gat, jnp.exp(gat) - 1.0)
    h4 = h3_ref[...] + elu
    logits = jnp.dot(h4, wo_ref[...], preferred_element_type=jnp.float32)
    logits = logits + bo_ref[...]
    mx = jnp.max(logits, axis=1, keepdims=True)
    z = logits - mx
    o_ref[...] = z - jnp.log(jnp.sum(jnp.exp(z), axis=1, keepdims=True))


def _tc_final(msg, g, h3, ex_self, denom, selt, bg, wout, bout):
    m = msg.shape[0]
    c = wout.shape[1]
    return pl.pallas_call(
        _final_body,
        grid=(m // _BM,),
        in_specs=[
            pl.BlockSpec((_BM, _H), lambda i: (i, 0)),
            pl.BlockSpec((_BM, _H), lambda i: (i, 0)),
            pl.BlockSpec((_BM, _H), lambda i: (i, 0)),
            pl.BlockSpec((_BM, _HEADS), lambda i: (i, 0)),
            pl.BlockSpec((_BM, _HEADS), lambda i: (i, 0)),
            pl.BlockSpec((_HEADS, _H), lambda i: (0, 0)),
            pl.BlockSpec((1, _H), lambda i: (0, 0)),
            pl.BlockSpec((_H, c), lambda i: (0, 0)),
            pl.BlockSpec((1, c), lambda i: (0, 0)),
        ],
        out_specs=pl.BlockSpec((_BM, c), lambda i: (i, 0)),
        out_shape=jax.ShapeDtypeStruct((m, c), jnp.float32),
    )(msg, g, h3, ex_self, denom, selt, bg.reshape(1, _H), wout,
      bout.reshape(1, c))


def _sc_mesh():
    return plsc.VectorSubcoreMesh(core_axis_name="c", subcore_axis_name="s")


def _sc_degree(dsthalf, ones128, zrows):
    @functools.partial(
        pl.kernel,
        out_type=jax.ShapeDtypeStruct((2, _DACC, 128), jnp.float32),
        mesh=_sc_mesh(),
        scratch_types=[
            pltpu.VMEM((80, 128), jnp.int32),
            pltpu.VMEM((128, 128), jnp.float32),
            pltpu.VMEM_SHARED((_DACC, 128), jnp.float32),
        ],
    )
    def body(dh_ref, ones_ref, z_ref, out_ref, didx, onesv, acc):
        core = lax.axis_index("c")
        sub = lax.axis_index("s")
        r0 = sub * _DRPT
        pltpu.sync_copy(dh_ref.at[core].at[sub], didx)
        pltpu.sync_copy(ones_ref, onesv)
        pltpu.sync_copy(z_ref.at[pl.ds(0, _DRPT)], acc.at[pl.ds(r0, _DRPT)])
        plsc.subcore_barrier()

        def step(b, carry):
            pltpu.sync_copy(onesv, acc.at[didx.at[b]], add=True)
            return carry

        lax.fori_loop(0, 80, step, 0)
        plsc.subcore_barrier()
        pltpu.sync_copy(acc.at[pl.ds(r0, _DRPT)],
                        out_ref.at[core].at[pl.ds(r0, _DRPT)])

    return body(dsthalf, ones128, zrows)


def _sc_propagate(ytab, src16, dsthalf, zrows):
    @functools.partial(
        pl.kernel,
        out_type=jax.ShapeDtypeStruct((_NCHUNK, 2, _DACC, 128), jnp.float32),
        mesh=_sc_mesh(),
        scratch_types=[
            pltpu.VMEM((80, 128), jnp.int32),
            pltpu.VMEM((80, 128), jnp.int32),
            pltpu.VMEM((128, 128), jnp.float32),
            pltpu.VMEM_SHARED((_DACC, 128), jnp.float32),
            pltpu.SemaphoreType.DMA,
        ],
    )
    def body(ytab_ref, src_ref, dh_ref, z_ref, out_ref, sidx, didx, rows,
             acc, sem):
        core = lax.axis_index("c")
        sub = lax.axis_index("s")
        r0 = sub * _DRPT
        pltpu.sync_copy(src_ref.at[sub], sidx)

        def add_offset(delta):
            def row(r, carry):
                for k in range(8):
                    sidx[r, pl.ds(k * 16, 16)] = (
                        sidx[r, pl.ds(k * 16, 16)] + delta)
                return carry
            lax.fori_loop(0, 80, row, 0)

        for cl in range(2):
            chunk = core * 2 + cl
            if cl == 0:
                add_offset(core * (2 * _NPAD))
            else:
                add_offset(jnp.int32(_NPAD))
            for h in range(2):
                pltpu.sync_copy(dh_ref.at[h].at[sub], didx)
                pltpu.sync_copy(z_ref.at[pl.ds(0, _DRPT)],
                                acc.at[pl.ds(r0, _DRPT)])
                plsc.subcore_barrier()

                def step(b, carry):
                    pltpu.async_copy(ytab_ref.at[sidx.at[b]], rows,
                                     sem).wait()
                    pltpu.sync_copy(rows, acc.at[didx.at[b]], add=True)
                    return carry

                lax.fori_loop(0, 80, step, 0)
                plsc.subcore_barrier()
                pltpu.sync_copy(acc.at[pl.ds(r0, _DRPT)],
                                out_ref.at[chunk].at[h].at[pl.ds(r0, _DRPT)])
                plsc.subcore_barrier()

    return body(ytab, src16, dsthalf, zrows)


def _sc_att(t1, t2, src32, dst32, zrows, mask16):
    @functools.partial(
        pl.kernel,
        out_type=jax.ShapeDtypeStruct((_EPAD, 128), jnp.float32),
        mesh=_sc_mesh(),
        scratch_types=[
            pltpu.VMEM((40, 128), jnp.int32),
            pltpu.VMEM((40, 128), jnp.int32),
            pltpu.VMEM((128, 128), jnp.float32),
            pltpu.VMEM((128, 128), jnp.float32),
            pltpu.VMEM((128, 128), jnp.float32),
            pltpu.VMEM((1, 16), jnp.float32),
            pltpu.SemaphoreType.DMA,
            pltpu.SemaphoreType.DMA,
        ],
    )
    def body(t1_ref, t2_ref, src_ref, dst_ref, z_ref, mask_ref,
             ex_out, sidx, didx, srows, drows, exrows, maskv, sem1, sem2):
        core = lax.axis_index("c")
        sub = lax.axis_index("s")
        tile = core * 16 + sub
        pltpu.sync_copy(src_ref.at[tile], sidx)
        pltpu.sync_copy(dst_ref.at[tile], didx)
        pltpu.sync_copy(mask_ref, maskv)
        pltpu.sync_copy(z_ref.at[pl.ds(0, 128)], exrows)

        def batch(b, carry):
            pltpu.async_copy(t1_ref.at[sidx.at[b]], srows, sem1).wait()
            pltpu.async_copy(t2_ref.at[didx.at[b]], drows, sem2).wait()

            def edge(e, carry2):
                v0 = srows[e, pl.ds(0, 16)] + drows[e, pl.ds(0, 16)]
                al = jnp.maximum(v0, 0.2 * v0)
                cvec = drows[e, pl.ds(16, 16)]
                exv = jnp.exp(al - cvec) * maskv[0, pl.ds(0, 16)]
                exrows[e, pl.ds(0, 16)] = exv
                return carry2

            lax.fori_loop(0, 128, edge, 0)
            pltpu.sync_copy(exrows,
                            ex_out.at[pl.ds(tile * 5120 + b * 128, 128)])
            return carry

        lax.fori_loop(0, 40, batch, 0)

    return body(t1, t2, src32, dst32, zrows, mask16)


def _sc_den(extab, dsthalf, zrows):
    @functools.partial(
        pl.kernel,
        out_type=jax.ShapeDtypeStruct((2, _DACC, 128), jnp.float32),
        mesh=_sc_mesh(),
        scratch_types=[
            pltpu.VMEM((80, 128), jnp.int32),
            pltpu.VMEM((128, 128), jnp.float32),
            pltpu.VMEM_SHARED((_DACC, 128), jnp.float32),
            pltpu.SemaphoreType.DMA,
        ],
    )
    def body(ex_ref, dh_ref, z_ref, out_ref, didx, exbuf, acc, sem):
        core = lax.axis_index("c")
        sub = lax.axis_index("s")
        r0 = sub * _DRPT
        pltpu.sync_copy(dh_ref.at[core].at[sub], didx)
        pltpu.sync_copy(z_ref.at[pl.ds(0, _DRPT)], acc.at[pl.ds(r0, _DRPT)])
        plsc.subcore_barrier()

        def step(b, carry):
            pltpu.async_copy(
                ex_ref.at[pl.ds(sub * 10240 + b * 128, 128)], exbuf,
                sem).wait()
            pltpu.sync_copy(exbuf, acc.at[didx.at[b]], add=True)
            return carry

        lax.fori_loop(0, 80, step, 0)
        plsc.subcore_barrier()
        pltpu.sync_copy(acc.at[pl.ds(r0, _DRPT)],
                        out_ref.at[core].at[pl.ds(r0, _DRPT)])

    return body(extab, dsthalf, zrows)


def _sc_msg(gtab, src16, dsthalf, extab, zrows):
    @functools.partial(
        pl.kernel,
        out_type=jax.ShapeDtypeStruct((_NCHUNK, 2, _DACC, 128), jnp.float32),
        mesh=_sc_mesh(),
        scratch_types=[
            pltpu.VMEM((80, 128), jnp.int32),
            pltpu.VMEM((80, 128), jnp.int32),
            pltpu.VMEM((128, 128), jnp.float32),
            pltpu.VMEM((128, 128), jnp.float32),
            pltpu.VMEM_SHARED((_DACC, 128), jnp.float32),
            pltpu.SemaphoreType.DMA,
            pltpu.SemaphoreType.DMA,
        ],
    )
    def body(gtab_ref, src_ref, dh_ref, ex_ref, z_ref, out_ref, sidx, didx,
             rows, exbuf, acc, sem1, sem2):
        core = lax.axis_index("c")
        sub = lax.axis_index("s")
        r0 = sub * _DRPT
        pltpu.sync_copy(src_ref.at[sub], sidx)

        def add_offset(delta):
            def row(r, carry):
                for k in range(8):
                    sidx[r, pl.ds(k * 16, 16)] = (
                        sidx[r, pl.ds(k * 16, 16)] + delta)
                return carry
            lax.fori_loop(0, 80, row, 0)

        for cl in range(2):
            chunk = core * 2 + cl
            if cl == 0:
                add_offset(core * (2 * _NPAD))
            else:
                add_offset(jnp.int32(_NPAD))
            for h in range(2):
                pltpu.sync_copy(dh_ref.at[h].at[sub], didx)
                pltpu.sync_copy(z_ref.at[pl.ds(0, _DRPT)],
                                acc.at[pl.ds(r0, _DRPT)])
                plsc.subcore_barrier()

                def step(b, carry):
                    pltpu.async_copy(gtab_ref.at[sidx.at[b]], rows,
                                     sem1).wait()
                    pltpu.async_copy(
                        ex_ref.at[pl.ds(sub * 10240 + b * 128, 128)], exbuf,
                        sem2).wait()

                    def edge(e, carry2):
                        s0 = exbuf[e, pl.ds(32 * chunk, 16)]
                        s1 = exbuf[e, pl.ds(32 * chunk + 16, 16)]
                        for k in range(4):
                            rows[e, pl.ds(k * 16, 16)] = (
                                rows[e, pl.ds(k * 16, 16)] * s0)
                        for k in range(4, 8):
                            rows[e, pl.ds(k * 16, 16)] = (
                                rows[e, pl.ds(k * 16, 16)] * s1)
                        return carry2

                    lax.fori_loop(0, 128, edge, 0)
                    pltpu.sync_copy(rows, acc.at[didx.at[b]], add=True)
                    return carry

                lax.fori_loop(0, 80, step, 0)
                plsc.subcore_barrier()
                pltpu.sync_copy(acc.at[pl.ds(r0, _DRPT)],
                                out_ref.at[chunk].at[h].at[pl.ds(r0, _DRPT)])
                plsc.subcore_barrier()

    return body(gtab, src16, dsthalf, extab, zrows)


def _chunked(a):
    ap = jnp.pad(a, ((0, _NPAD - _N), (0, 0)))
    return ap.reshape(_NPAD, _NCHUNK, 128).transpose(1, 0, 2).reshape(
        _NCHUNK * _NPAD, 128)


def _unchunked(t):
    full = jnp.concatenate([t[:, 0, :_DH], t[:, 1, :_DH]], axis=1)
    return full.transpose(1, 0, 2).reshape(2 * _DH, _H)[:_N]


def kernel(x, edge_index, W1, b1, Wm, bm, W2, b2, Wg, att_src, att_dst, bg,
           Wout, bout):
    src = edge_index[0]
    dst = edge_index[1]
    src_p = jnp.pad(src, (0, _EPAD - _E))
    dst_p = jnp.pad(dst, (0, _EPAD - _E),
                    constant_values=_N)
    src16 = src_p.reshape(16, 80, 128)
    src32 = src_p.reshape(32, 40, 128)
    dst32 = dst_p.reshape(32, 40, 128)
    dh0 = jnp.where(dst_p < _DH, dst_p, _DH)
    dh1 = jnp.where(dst_p >= _DH, dst_p - _DH, _DH)
    dsthalf = jnp.stack([dh0, dh1]).reshape(2, 16, 80, 128)
    zrows = jnp.zeros((_DRPT, 128), jnp.float32)
    ones128 = jnp.ones((128, 128), jnp.float32)
    mask16 = jnp.zeros((1, 16), jnp.float32).at[0, 0:8].set(1.0)

    degs = _sc_degree(dsthalf, ones128, zrows)
    deg = jnp.concatenate([degs[0, :_DH], degs[1, :_DH]])[:_N, 0] + 1.0
    dis = lax.rsqrt(deg).reshape(_N, 1)

    y1 = _tc_scale_mm(x, W1, dis)
    acc1 = _unchunked(_sc_propagate(_chunked(y1), src16, dsthalf, zrows))
    y2, _ = _tc_chain(acc1, y1, dis, b1, Wm)
    acc2 = _unchunked(_sc_propagate(_chunked(y2), src16, dsthalf, zrows))
    y3, _ = _tc_chain(acc2, y2, dis, bm, W2)
    acc3 = _unchunked(_sc_propagate(_chunked(y3), src16, dsthalf, zrows))

    eye = jnp.eye(_HEADS, dtype=jnp.float32)
    sel = jnp.repeat(eye, _FH, axis=0)
    asf = att_src.reshape(1, _H)
    adf = att_dst.reshape(1, _H)
    h3, g, a_src, a_dst = _tc_gat_prep(acc3, y3, dis, b2, Wg, asf, adf, sel)

    gmax = a_src.max(axis=0)
    cdst = a_dst + gmax[None]
    cdst = jnp.where(cdst > 0, cdst, 0.2 * cdst)

    zpadn = ((0, _NPAD - _N), (0, 0))
    t1 = jnp.pad(jnp.concatenate([a_src, a_src, jnp.zeros((_N, 112))],
                                 axis=1).astype(jnp.float32), zpadn)
    t2 = jnp.pad(jnp.concatenate([a_dst, a_dst, cdst, jnp.zeros((_N, 104))],
                                 axis=1).astype(jnp.float32), zpadn)

    extab = _sc_att(t1, t2, src32, dst32, zrows, mask16)
    dens = _sc_den(extab, dsthalf, zrows)
    denom = jnp.concatenate([dens[0, :_DH], dens[1, :_DH]])[:_N, 0:8]
    aself = a_src + a_dst
    aself = jnp.where(aself > 0, aself, 0.2 * aself)
    ex_self = jnp.exp(aself - cdst)
    denom = denom + ex_self + 1e-16

    exsplat = jnp.repeat(extab[:, 0:8], 16, axis=1)
    msg = _unchunked(_sc_msg(_chunked(g), src16, dsthalf, exsplat, zrows))

    return _tc_final(msg, g, h3, ex_self, denom, sel.T, bg, Wout, bout)


# speedup vs baseline: 5.0162x; 1.2030x over previous
import functools
import jax
import jax.numpy as jnp
from jax import lax
from jax.experimental import pallas as pl
from jax.experimental.pallas import tpu as pltpu
from jax.experimental.pallas import tpu_sc as plsc

_N = 10000
_NPAD = 10112
_E = 160000
_EPAD = 163840
_HEADS = 8
_FH = 64
_H = 512
_NCHUNK = 4
_DH = 5056
_DACC = 5120
_DRPT = _DACC // 16
_BM = 400


def _mm_scale_body(x_ref, w_ref, s_ref, o_ref):
    acc = jnp.dot(x_ref[...], w_ref[...], preferred_element_type=jnp.float32)
    o_ref[...] = acc * s_ref[...]


def _tc_scale_mm(x, w, scale):
    m, k = x.shape
    n = w.shape[1]
    return pl.pallas_call(
        _mm_scale_body,
        grid=(m // _BM,),
        in_specs=[
            pl.BlockSpec((_BM, k), lambda i: (i, 0)),
            pl.BlockSpec((k, n), lambda i: (0, 0)),
            pl.BlockSpec((_BM, 1), lambda i: (i, 0)),
        ],
        out_specs=pl.BlockSpec((_BM, n), lambda i: (i, 0)),
        out_shape=jax.ShapeDtypeStruct((m, n), jnp.float32),
    )(x, w, scale)


def _chain_body(acc_ref, y_ref, s_ref, b_ref, w_ref, o_ref, h_ref):
    h = jnp.maximum(s_ref[...] * (acc_ref[...] + y_ref[...]) + b_ref[...], 0.0)
    h_ref[...] = h
    o_ref[...] = s_ref[...] * jnp.dot(h, w_ref[...],
                                      preferred_element_type=jnp.float32)


def _tc_chain(acc, y, scale, b, w):
    m, k = y.shape
    n = w.shape[1]
    return pl.pallas_call(
        _chain_body,
        grid=(m // _BM,),
        in_specs=[
            pl.BlockSpec((_BM, k), lambda i: (i, 0)),
            pl.BlockSpec((_BM, k), lambda i: (i, 0)),
            pl.BlockSpec((_BM, 1), lambda i: (i, 0)),
            pl.BlockSpec((1, k), lambda i: (0, 0)),
            pl.BlockSpec((k, n), lambda i: (0, 0)),
        ],
        out_specs=[
            pl.BlockSpec((_BM, n), lambda i: (i, 0)),
            pl.BlockSpec((_BM, k), lambda i: (i, 0)),
        ],
        out_shape=[
            jax.ShapeDtypeStruct((m, n), jnp.float32),
            jax.ShapeDtypeStruct((m, k), jnp.float32),
        ],
    )(acc, y, scale, b.reshape(1, k), w)


def _gatprep_body(acc_ref, y_ref, s_ref, b_ref, wg_ref, asf_ref, adf_ref,
                  sel_ref, h_ref, g_ref, as_ref, ad_ref):
    h = jnp.maximum(s_ref[...] * (acc_ref[...] + y_ref[...]) + b_ref[...], 0.0)
    h_ref[...] = h
    g = jnp.dot(h, wg_ref[...], preferred_element_type=jnp.float32)
    g_ref[...] = g
    as_ref[...] = jnp.dot(g * asf_ref[...], sel_ref[...],
                          preferred_element_type=jnp.float32)
    ad_ref[...] = jnp.dot(g * adf_ref[...], sel_ref[...],
                          preferred_element_type=jnp.float32)


def _tc_gat_prep(acc, y, scale, b, wg, att_src_flat, att_dst_flat, sel):
    m, k = y.shape
    return pl.pallas_call(
        _gatprep_body,
        grid=(m // _BM,),
        in_specs=[
            pl.BlockSpec((_BM, k), lambda i: (i, 0)),
            pl.BlockSpec((_BM, k), lambda i: (i, 0)),
            pl.BlockSpec((_BM, 1), lambda i: (i, 0)),
            pl.BlockSpec((1, k), lambda i: (0, 0)),
            pl.BlockSpec((k, _H), lambda i: (0, 0)),
            pl.BlockSpec((1, _H), lambda i: (0, 0)),
            pl.BlockSpec((1, _H), lambda i: (0, 0)),
            pl.BlockSpec((_H, _HEADS), lambda i: (0, 0)),
        ],
        out_specs=[
            pl.BlockSpec((_BM, k), lambda i: (i, 0)),
            pl.BlockSpec((_BM, _H), lambda i: (i, 0)),
            pl.BlockSpec((_BM, _HEADS), lambda i: (i, 0)),
            pl.BlockSpec((_BM, _HEADS), lambda i: (i, 0)),
        ],
        out_shape=[
            jax.ShapeDtypeStruct((m, k), jnp.float32),
            jax.ShapeDtypeStruct((m, _H), jnp.float32),
            jax.ShapeDtypeStruct((m, _HEADS), jnp.float32),
            jax.ShapeDtypeStruct((m, _HEADS), jnp.float32),
        ],
    )(acc, y, scale, b.reshape(1, k), wg, att_src_flat, att_dst_flat, sel)


def _final_body(msg_ref, g_ref, h3_ref, exs_ref, den_ref, selt_ref, bg_ref,
                wo_ref, bo_ref, o_ref):
    exf = jnp.dot(exs_ref[...], selt_ref[...],
                  preferred_element_type=jnp.float32)
    denf = jnp.dot(den_ref[...], selt_ref[...],
                   preferred_element_type=jnp.float32)
    gat = (msg_ref[...] + g_ref[...] * exf) / denf + bg_ref[...]
    elu = jnp.where(gat > 0, gat, jnp.exp(gat) - 1.0)
    h4 = h3_ref[...] + elu
    logits = jnp.dot(h4, wo_ref[...], preferred_element_type=jnp.float32)
    logits = logits + bo_ref[...]
    mx = jnp.max(logits, axis=1, keepdims=True)
    z = logits - mx
    o_ref[...] = z - jnp.log(jnp.sum(jnp.exp(z), axis=1, keepdims=True))


def _tc_final(msg, g, h3, ex_self, denom, selt, bg, wout, bout):
    m = msg.shape[0]
    c = wout.shape[1]
    return pl.pallas_call(
        _final_body,
        grid=(m // _BM,),
        in_specs=[
            pl.BlockSpec((_BM, _H), lambda i: (i, 0)),
            pl.BlockSpec((_BM, _H), lambda i: (i, 0)),
            pl.BlockSpec((_BM, _H), lambda i: (i, 0)),
            pl.BlockSpec((_BM, _HEADS), lambda i: (i, 0)),
            pl.BlockSpec((_BM, _HEADS), lambda i: (i, 0)),
            pl.BlockSpec((_HEADS, _H), lambda i: (0, 0)),
            pl.BlockSpec((1, _H), lambda i: (0, 0)),
            pl.BlockSpec((_H, c), lambda i: (0, 0)),
            pl.BlockSpec((1, c), lambda i: (0, 0)),
        ],
        out_specs=pl.BlockSpec((_BM, c), lambda i: (i, 0)),
        out_shape=jax.ShapeDtypeStruct((m, c), jnp.float32),
    )(msg, g, h3, ex_self, denom, selt, bg.reshape(1, _H), wout,
      bout.reshape(1, c))


def _sc_mesh():
    return plsc.VectorSubcoreMesh(core_axis_name="c", subcore_axis_name="s")


def _sc_degree(dsthalf, ones128, zrows):
    @functools.partial(
        pl.kernel,
        out_type=jax.ShapeDtypeStruct((2, _DACC, 128), jnp.float32),
        mesh=_sc_mesh(),
        scratch_types=[
            pltpu.VMEM((80, 128), jnp.int32),
            pltpu.VMEM((128, 128), jnp.float32),
            pltpu.VMEM_SHARED((_DACC, 128), jnp.float32),
        ],
    )
    def body(dh_ref, ones_ref, z_ref, out_ref, didx, onesv, acc):
        core = lax.axis_index("c")
        sub = lax.axis_index("s")
        r0 = sub * _DRPT
        pltpu.sync_copy(dh_ref.at[core].at[sub], didx)
        pltpu.sync_copy(ones_ref, onesv)
        pltpu.sync_copy(z_ref.at[pl.ds(0, _DRPT)], acc.at[pl.ds(r0, _DRPT)])
        plsc.subcore_barrier()

        def step(b, carry):
            pltpu.sync_copy(onesv, acc.at[didx.at[b]], add=True)
            return carry

        lax.fori_loop(0, 80, step, 0)
        plsc.subcore_barrier()
        pltpu.sync_copy(acc.at[pl.ds(r0, _DRPT)],
                        out_ref.at[core].at[pl.ds(r0, _DRPT)])

    return body(dsthalf, ones128, zrows)


def _sc_propagate(ytab, src16, dsthalf, zrows):
    @functools.partial(
        pl.kernel,
        out_type=jax.ShapeDtypeStruct((_NCHUNK, 2, _DACC, 128), jnp.float32),
        mesh=_sc_mesh(),
        scratch_types=[
            pltpu.VMEM((80, 128), jnp.int32),
            pltpu.VMEM((80, 128), jnp.int32),
            pltpu.VMEM((128, 128), jnp.float32),
            pltpu.VMEM((128, 128), jnp.float32),
            pltpu.VMEM_SHARED((_DACC, 128), jnp.float32),
            pltpu.SemaphoreType.DMA,
            pltpu.SemaphoreType.DMA,
        ],
    )
    def body(ytab_ref, src_ref, dh_ref, z_ref, out_ref, sidx, didx, rows0,
             rows1, acc, sem0, sem1):
        core = lax.axis_index("c")
        sub = lax.axis_index("s")
        r0 = sub * _DRPT
        pltpu.sync_copy(src_ref.at[sub], sidx)

        def add_offset(delta):
            def row(r, carry):
                for k in range(8):
                    sidx[r, pl.ds(k * 16, 16)] = (
                        sidx[r, pl.ds(k * 16, 16)] + delta)
                return carry
            lax.fori_loop(0, 80, row, 0)

        for cl in range(2):
            chunk = core * 2 + cl
            if cl == 0:
                add_offset(core * (2 * _NPAD))
            else:
                add_offset(jnp.int32(_NPAD))
            for h in range(2):
                pltpu.sync_copy(dh_ref.at[h].at[sub], didx)
                pltpu.sync_copy(z_ref.at[pl.ds(0, _DRPT)],
                                acc.at[pl.ds(r0, _DRPT)])
                plsc.subcore_barrier()
                pltpu.async_copy(ytab_ref.at[sidx.at[0]], rows0, sem0)

                def pair(i, carry):
                    b0 = 2 * i
                    b1 = b0 + 1
                    pltpu.async_copy(ytab_ref.at[sidx.at[b1]], rows1, sem1)
                    pltpu.make_async_copy(ytab_ref.at[sidx.at[b0]], rows0,
                                          sem0).wait()
                    pltpu.sync_copy(rows0, acc.at[didx.at[b0]], add=True)

                    @pl.when(i < 39)
                    def _():
                        pltpu.async_copy(ytab_ref.at[sidx.at[b0 + 2]],
                                         rows0, sem0)

                    pltpu.make_async_copy(ytab_ref.at[sidx.at[b1]], rows1,
                                          sem1).wait()
                    pltpu.sync_copy(rows1, acc.at[didx.at[b1]], add=True)
                    return carry

                lax.fori_loop(0, 40, pair, 0)
                plsc.subcore_barrier()
                pltpu.sync_copy(acc.at[pl.ds(r0, _DRPT)],
                                out_ref.at[chunk].at[h].at[pl.ds(r0, _DRPT)])
                plsc.subcore_barrier()

    return body(ytab, src16, dsthalf, zrows)


def _sc_att(t1, t2, src32, dst32, zrows, mask16):
    @functools.partial(
        pl.kernel,
        out_type=jax.ShapeDtypeStruct((_EPAD, 128), jnp.float32),
        mesh=_sc_mesh(),
        scratch_types=[
            pltpu.VMEM((40, 128), jnp.int32),
            pltpu.VMEM((40, 128), jnp.int32),
            pltpu.VMEM((128, 128), jnp.float32),
            pltpu.VMEM((128, 128), jnp.float32),
            pltpu.VMEM((128, 128), jnp.float32),
            pltpu.VMEM((1, 16), jnp.float32),
            pltpu.SemaphoreType.DMA,
            pltpu.SemaphoreType.DMA,
        ],
    )
    def body(t1_ref, t2_ref, src_ref, dst_ref, z_ref, mask_ref,
             ex_out, sidx, didx, srows, drows, exrows, maskv, sem1, sem2):
        core = lax.axis_index("c")
        sub = lax.axis_index("s")
        tile = core * 16 + sub
        pltpu.sync_copy(src_ref.at[tile], sidx)
        pltpu.sync_copy(dst_ref.at[tile], didx)
        pltpu.sync_copy(mask_ref, maskv)
        pltpu.sync_copy(z_ref.at[pl.ds(0, 128)], exrows)

        def batch(b, carry):
            pltpu.async_copy(t1_ref.at[sidx.at[b]], srows, sem1).wait()
            pltpu.async_copy(t2_ref.at[didx.at[b]], drows, sem2).wait()

            def edge(e, carry2):
                v0 = srows[e, pl.ds(0, 16)] + drows[e, pl.ds(0, 16)]
                al = jnp.maximum(v0, 0.2 * v0)
                cvec = drows[e, pl.ds(16, 16)]
                exv = jnp.exp(al - cvec) * maskv[0, pl.ds(0, 16)]
                exrows[e, pl.ds(0, 16)] = exv
                return carry2

            lax.fori_loop(0, 128, edge, 0)
            pltpu.sync_copy(exrows,
                            ex_out.at[pl.ds(tile * 5120 + b * 128, 128)])
            return carry

        lax.fori_loop(0, 40, batch, 0)

    return body(t1, t2, src32, dst32, zrows, mask16)


def _sc_den(extab, dsthalf, zrows):
    @functools.partial(
        pl.kernel,
        out_type=jax.ShapeDtypeStruct((2, _DACC, 128), jnp.float32),
        mesh=_sc_mesh(),
        scratch_types=[
            pltpu.VMEM((80, 128), jnp.int32),
            pltpu.VMEM((128, 128), jnp.float32),
            pltpu.VMEM_SHARED((_DACC, 128), jnp.float32),
            pltpu.SemaphoreType.DMA,
        ],
    )
    def body(ex_ref, dh_ref, z_ref, out_ref, didx, exbuf, acc, sem):
        core = lax.axis_index("c")
        sub = lax.axis_index("s")
        r0 = sub * _DRPT
        pltpu.sync_copy(dh_ref.at[core].at[sub], didx)
        pltpu.sync_copy(z_ref.at[pl.ds(0, _DRPT)], acc.at[pl.ds(r0, _DRPT)])
        plsc.subcore_barrier()

        def step(b, carry):
            pltpu.async_copy(
                ex_ref.at[pl.ds(sub * 10240 + b * 128, 128)], exbuf,
                sem).wait()
            pltpu.sync_copy(exbuf, acc.at[didx.at[b]], add=True)
            return carry

        lax.fori_loop(0, 80, step, 0)
        plsc.subcore_barrier()
        pltpu.sync_copy(acc.at[pl.ds(r0, _DRPT)],
                        out_ref.at[core].at[pl.ds(r0, _DRPT)])

    return body(extab, dsthalf, zrows)


def _sc_msg(gtab, src16, dsthalf, extab, zrows):
    @functools.partial(
        pl.kernel,
        out_type=jax.ShapeDtypeStruct((_NCHUNK, 2, _DACC, 128), jnp.float32),
        mesh=_sc_mesh(),
        scratch_types=[
            pltpu.VMEM((80, 128), jnp.int32),
            pltpu.VMEM((80, 128), jnp.int32),
            pltpu.VMEM((128, 128), jnp.float32),
            pltpu.VMEM((128, 128), jnp.float32),
            pltpu.VMEM((128, 128), jnp.float32),
            pltpu.VMEM((128, 128), jnp.float32),
            pltpu.VMEM_SHARED((_DACC, 128), jnp.float32),
            pltpu.SemaphoreType.DMA,
            pltpu.SemaphoreType.DMA,
            pltpu.SemaphoreType.DMA,
            pltpu.SemaphoreType.DMA,
        ],
    )
    def body(gtab_ref, src_ref, dh_ref, ex_ref, z_ref, out_ref, sidx, didx,
             rows0, rows1, exbuf0, exbuf1, acc, sem0, sem1, sem2, sem3):
        core = lax.axis_index("c")
        sub = lax.axis_index("s")
        r0 = sub * _DRPT
        pltpu.sync_copy(src_ref.at[sub], sidx)

        def add_offset(delta):
            def row(r, carry):
                for k in range(8):
                    sidx[r, pl.ds(k * 16, 16)] = (
                        sidx[r, pl.ds(k * 16, 16)] + delta)
                return carry
            lax.fori_loop(0, 80, row, 0)

        for cl in range(2):
            chunk = core * 2 + cl
            if cl == 0:
                add_offset(core * (2 * _NPAD))
            else:
                add_offset(jnp.int32(_NPAD))
            for h in range(2):
                pltpu.sync_copy(dh_ref.at[h].at[sub], didx)
                pltpu.sync_copy(z_ref.at[pl.ds(0, _DRPT)],
                                acc.at[pl.ds(r0, _DRPT)])
                plsc.subcore_barrier()
                pltpu.async_copy(gtab_ref.at[sidx.at[0]], rows0, sem0)
                pltpu.async_copy(ex_ref.at[pl.ds(sub * 10240, 128)],
                                 exbuf0, sem2)

                def mul_scatter(b, rows, exbuf):
                    def edge(e, carry2):
                        s0 = exbuf[e, pl.ds(32 * chunk, 16)]
                        s1 = exbuf[e, pl.ds(32 * chunk + 16, 16)]
                        for k in range(4):
                            rows[e, pl.ds(k * 16, 16)] = (
                                rows[e, pl.ds(k * 16, 16)] * s0)
                        for k in range(4, 8):
                            rows[e, pl.ds(k * 16, 16)] = (
                                rows[e, pl.ds(k * 16, 16)] * s1)
                        return carry2

                    lax.fori_loop(0, 128, edge, 0)
                    pltpu.sync_copy(rows, acc.at[didx.at[b]], add=True)

                def pair(i, carry):
                    b0 = 2 * i
                    b1 = b0 + 1
                    pltpu.async_copy(gtab_ref.at[sidx.at[b1]], rows1, sem1)
                    pltpu.async_copy(
                        ex_ref.at[pl.ds(sub * 10240 + b1 * 128, 128)],
                        exbuf1, sem3)
                    pltpu.make_async_copy(gtab_ref.at[sidx.at[b0]], rows0,
                                          sem0).wait()
                    pltpu.make_async_copy(
                        ex_ref.at[pl.ds(sub * 10240, 128)], exbuf0,
                        sem2).wait()
                    mul_scatter(b0, rows0, exbuf0)

                    @pl.when(i < 39)
                    def _():
                        pltpu.async_copy(gtab_ref.at[sidx.at[b0 + 2]],
                                         rows0, sem0)
                        pltpu.async_copy(
                            ex_ref.at[pl.ds(sub * 10240 + (b0 + 2) * 128,
                                            128)], exbuf0, sem2)

                    pltpu.make_async_copy(gtab_ref.at[sidx.at[b1]], rows1,
                                          sem1).wait()
                    pltpu.make_async_copy(
                        ex_ref.at[pl.ds(sub * 10240, 128)], exbuf1,
                        sem3).wait()
                    mul_scatter(b1, rows1, exbuf1)
                    return carry

                lax.fori_loop(0, 40, pair, 0)
                plsc.subcore_barrier()
                pltpu.sync_copy(acc.at[pl.ds(r0, _DRPT)],
                                out_ref.at[chunk].at[h].at[pl.ds(r0, _DRPT)])
                plsc.subcore_barrier()

    return body(gtab, src16, dsthalf, extab, zrows)


def _chunked(a):
    ap = jnp.pad(a, ((0, _NPAD - _N), (0, 0)))
    return ap.reshape(_NPAD, _NCHUNK, 128).transpose(1, 0, 2).reshape(
        _NCHUNK * _NPAD, 128)


def _unchunked(t):
    full = jnp.concatenate([t[:, 0, :_DH], t[:, 1, :_DH]], axis=1)
    return full.transpose(1, 0, 2).reshape(2 * _DH, _H)[:_N]


def kernel(x, edge_index, W1, b1, Wm, bm, W2, b2, Wg, att_src, att_dst, bg,
           Wout, bout):
    src = edge_index[0]
    dst = edge_index[1]
    src_p = jnp.pad(src, (0, _EPAD - _E))
    dst_p = jnp.pad(dst, (0, _EPAD - _E),
                    constant_values=_N)
    src16 = src_p.reshape(16, 80, 128)
    src32 = src_p.reshape(32, 40, 128)
    dst32 = dst_p.reshape(32, 40, 128)
    dh0 = jnp.where(dst_p < _DH, dst_p, _DH)
    dh1 = jnp.where(dst_p >= _DH, dst_p - _DH, _DH)
    dsthalf = jnp.stack([dh0, dh1]).reshape(2, 16, 80, 128)
    zrows = jnp.zeros((_DRPT, 128), jnp.float32)
    ones128 = jnp.ones((128, 128), jnp.float32)
    mask16 = jnp.zeros((1, 16), jnp.float32).at[0, 0:8].set(1.0)

    degs = _sc_degree(dsthalf, ones128, zrows)
    deg = jnp.concatenate([degs[0, :_DH], degs[1, :_DH]])[:_N, 0] + 1.0
    dis = lax.rsqrt(deg).reshape(_N, 1)

    y1 = _tc_scale_mm(x, W1, dis)
    acc1 = _unchunked(_sc_propagate(_chunked(y1), src16, dsthalf, zrows))
    y2, _ = _tc_chain(acc1, y1, dis, b1, Wm)
    acc2 = _unchunked(_sc_propagate(_chunked(y2), src16, dsthalf, zrows))
    y3, _ = _tc_chain(acc2, y2, dis, bm, W2)
    acc3 = _unchunked(_sc_propagate(_chunked(y3), src16, dsthalf, zrows))

    eye = jnp.eye(_HEADS, dtype=jnp.float32)
    sel = jnp.repeat(eye, _FH, axis=0)
    asf = att_src.reshape(1, _H)
    adf = att_dst.reshape(1, _H)
    h3, g, a_src, a_dst = _tc_gat_prep(acc3, y3, dis, b2, Wg, asf, adf, sel)

    gmax = a_src.max(axis=0)
    cdst = a_dst + gmax[None]
    cdst = jnp.where(cdst > 0, cdst, 0.2 * cdst)

    zpadn = ((0, _NPAD - _N), (0, 0))
    t1 = jnp.pad(jnp.concatenate([a_src, a_src, jnp.zeros((_N, 112))],
                                 axis=1).astype(jnp.float32), zpadn)
    t2 = jnp.pad(jnp.concatenate([a_dst, a_dst, cdst, jnp.zeros((_N, 104))],
                                 axis=1).astype(jnp.float32), zpadn)

    extab = _sc_att(t1, t2, src32, dst32, zrows, mask16)
    dens = _sc_den(extab, dsthalf, zrows)
    denom = jnp.concatenate([dens[0, :_DH], dens[1, :_DH]])[:_N, 0:8]
    aself = a_src + a_dst
    aself = jnp.where(aself > 0, aself, 0.2 * aself)
    ex_self = jnp.exp(aself - cdst)
    denom = denom + ex_self + 1e-16

    exsplat = jnp.repeat(extab[:, 0:8], 16, axis=1)
    msg = _unchunked(_sc_msg(_chunked(g), src16, dsthalf, exsplat, zrows))

    return _tc_final(msg, g, h3, ex_self, denom, sel.T, bg, Wout, bout)
